# baseline (speedup 1.0000x reference)
import jax
import jax.numpy as jnp
from jax.experimental import pallas as pl
from jax.experimental.pallas import tpu as pltpu

_MATCH_THRESH = 0.5
_NEG_POS = 3.0
_VAR0 = 0.1
_VAR1 = 0.2


def _per_image_kernel(targets_ref, anchors_ref, pred_loc_ref, pred_conf_ref,
                      ce_ref, stats_ref):
    for img in range(targets_ref.shape[0]):
        _one_image(img, targets_ref, anchors_ref, pred_loc_ref, pred_conf_ref,
                   ce_ref, stats_ref)


def _one_image(img, targets_ref, anchors_ref, pred_loc_ref, pred_conf_ref,
               ce_ref, stats_ref):
    tgt = targets_ref[img]
    tx0 = tgt[:, 0:1]
    ty0 = tgt[:, 1:2]
    tx1 = tgt[:, 2:3]
    ty1 = tgt[:, 3:4]
    tlab = tgt[:, 4:5]

    acx = anchors_ref[0, 0:1, :]
    acy = anchors_ref[0, 1:2, :]
    aw = anchors_ref[0, 2:3, :]
    ah = anchors_ref[0, 3:4, :]
    px0 = acx - aw * 0.5
    py0 = acy - ah * 0.5
    px1 = acx + aw * 0.5
    py1 = acy + ah * 0.5

    ix = jnp.minimum(tx1, px1) - jnp.maximum(tx0, px0)
    iy = jnp.minimum(ty1, py1) - jnp.maximum(ty0, py0)
    inter = jnp.maximum(ix, 0.0) * jnp.maximum(iy, 0.0)
    area_t = (tx1 - tx0) * (ty1 - ty0)
    area_p = (px1 - px0) * (py1 - py0)
    ovl = inter / (area_t + area_p - inter)

    n_t, A = ovl.shape
    j_iota = jax.lax.broadcasted_iota(jnp.int32, (n_t, A), 0)
    a_iota = jax.lax.broadcasted_iota(jnp.int32, (n_t, A), 1)
    lane_a = jax.lax.broadcasted_iota(jnp.int32, (1, A), 1)

    bto = jnp.max(ovl, axis=0, keepdims=True)
    bti = jnp.min(jnp.where(ovl == bto, j_iota, n_t),
                  axis=0, keepdims=True)
    rmax = jnp.max(ovl, axis=1, keepdims=True)
    bpi = jnp.min(jnp.where(ovl == rmax, a_iota, A),
                  axis=1, keepdims=True)

    m_claim = lane_a == bpi
    chosen = jnp.max(jnp.where(m_claim, j_iota, -1), axis=0,
                     keepdims=True)
    claimed = chosen >= 0
    bto = jnp.where(claimed, 2.0, bto)
    bti = jnp.where(claimed, chosen, bti)

    m_sel = (j_iota == bti).astype(jnp.float32)
    tgt_t = jnp.transpose(tgt, (1, 0))
    matched = jax.lax.dot_general(
        tgt_t, m_sel, (((1,), (0,)), ((), ())),
        preferred_element_type=jnp.float32)
    mx0 = matched[0:1, :]
    my0 = matched[1:2, :]
    mx1 = matched[2:3, :]
    my1 = matched[3:4, :]
    mlab = matched[4:5, :]
    conf = jnp.where(bto < _MATCH_THRESH, 0.0, mlab + 1.0)
    pos = conf > 0.0
    pf = pos.astype(jnp.float32)

    gcx = ((mx0 + mx1) * 0.5 - acx) / (_VAR0 * aw)
    gcy = ((my0 + my1) * 0.5 - acy) / (_VAR0 * ah)
    gw = jnp.log((mx1 - mx0) / aw) / _VAR1
    gh = jnp.log((my1 - my0) / ah) / _VAR1

    lloss = jnp.float32(0.0)
    for i, gc in enumerate((gcx, gcy, gw, gh)):
        d = pred_loc_ref[img, i:i + 1, :] - gc
        ad = jnp.abs(d)
        sl1 = jnp.where(ad < 1.0, 0.5 * d * d, ad - 0.5)
        lloss = lloss + jnp.sum(sl1 * pf)
    npos = jnp.sum(pf)

    xt = pred_conf_ref[:, img, :]
    C = xt.shape[0]
    se_row = jnp.sum(jnp.exp(xt), axis=0, keepdims=True)
    cls_sub = jax.lax.broadcasted_iota(jnp.int32, xt.shape, 0)
    conf_i = conf.astype(jnp.int32)
    xg_row = jnp.sum(jnp.where(cls_sub == conf_i, xt, 0.0),
                     axis=0, keepdims=True)
    ce_row = jnp.log(se_row) - xg_row

    pos_sum = jnp.sum(jnp.where(pos, ce_row, 0.0))
    ce_ref[img] = jnp.where(pos, 0.0, ce_row)

    lane128 = jax.lax.broadcasted_iota(jnp.int32, (1, 128), 1)
    stats = jnp.where(lane128 == 0, npos,
                      jnp.where(lane128 == 1, pos_sum,
                                jnp.where(lane128 == 2, lloss, 0.0)))
    stats_ref[img] = stats


def _finalize_kernel(ce_ref, stats_ref, out_ref):
    ce = ce_ref[:, 0, :]
    stats = stats_ref[:, 0, :]
    npos = stats[:, 0:1]
    pos_sum = stats[:, 1:2]
    lloss = stats[:, 2:3]
    A = ce.shape[1]

    k = jnp.maximum(10.0, jnp.minimum(npos * _NEG_POS, A - npos))

    bits = jax.lax.bitcast_convert_type(ce, jnp.int32)
    key = jnp.where(bits < 0, jnp.int32(-2147483648) - bits, bits)

    lo = jnp.min(key, axis=1, keepdims=True) - 1
    hi = jnp.max(key, axis=1, keepdims=True)

    CH = 128
    nfull = A // CH
    ntail = A - nfull * CH

    def _tree_sum(sel):
        pieces = [sel[:, i * CH:(i + 1) * CH] for i in range(nfull)]
        while len(pieces) > 1:
            nxt = [pieces[i] + pieces[i + 1]
                   for i in range(0, len(pieces) - 1, 2)]
            if len(pieces) % 2:
                nxt.append(pieces[-1])
            pieces = nxt
        total = jnp.sum(pieces[0], axis=1, keepdims=True)
        if ntail:
            total = total + jnp.sum(sel[:, nfull * CH:], axis=1,
                                    keepdims=True)
        return total

    def count_gt(mid):
        return _tree_sum(jnp.where(key > mid, 1.0, 0.0))

    for _ in range(32):
        mid = lo + ((hi - lo) >> 1)
        c = count_gt(mid)
        ge = c >= k
        lo = jnp.where(ge, mid, lo)
        hi = jnp.where(ge, hi, mid)
    cnt = count_gt(hi)
    s = _tree_sum(jnp.where(key > hi, ce, 0.0))
    vkb = jnp.where(hi < 0, jnp.int32(-2147483648) - hi, hi)
    vk = jax.lax.bitcast_convert_type(vkb, jnp.float32)
    neg_sum = s + (k - cnt) * vk

    class_loss = jnp.sum(pos_sum + neg_sum)
    n_total = jnp.sum(npos)
    l_total = jnp.sum(lloss)

    lane128 = jax.lax.broadcasted_iota(jnp.int32, (1, 128), 1)
    out = jnp.where(lane128 == 0, class_loss / n_total,
                    jnp.where(lane128 == 1, l_total / n_total,
                              jnp.where(lane128 == 2, n_total, 0.0)))
    out_ref[...] = out


def kernel(pred_conf, pred_loc, anchors, targets):
    B, A, C = pred_conf.shape
    n_obj = targets.shape[1]
    anchors_t = jnp.transpose(anchors, (0, 2, 1))
    pred_loc_t = jnp.transpose(pred_loc, (0, 2, 1))
    pred_conf_t = jnp.transpose(pred_conf, (2, 0, 1))

    ce_neg, stats = pl.pallas_call(
        _per_image_kernel,
        grid=(B // 8,),
        in_specs=[
            pl.BlockSpec((8, n_obj, 5), lambda b: (b, 0, 0)),
            pl.BlockSpec((1, 4, A), lambda b: (0, 0, 0)),
            pl.BlockSpec((8, 4, A), lambda b: (b, 0, 0)),
            pl.BlockSpec((C, 8, A), lambda b: (0, b, 0)),
        ],
        out_specs=[
            pl.BlockSpec((8, 1, A), lambda b: (b, 0, 0)),
            pl.BlockSpec((8, 1, 128), lambda b: (b, 0, 0)),
        ],
        out_shape=[
            jax.ShapeDtypeStruct((B, 1, A), jnp.float32),
            jax.ShapeDtypeStruct((B, 1, 128), jnp.float32),
        ],
        compiler_params=pltpu.CompilerParams(
            vmem_limit_bytes=100 * 1024 * 1024),
    )(targets, anchors_t, pred_loc_t, pred_conf_t)

    out = pl.pallas_call(
        _finalize_kernel,
        out_shape=jax.ShapeDtypeStruct((1, 128), jnp.float32),
    )(ce_neg, stats)

    return (out[0, 0], out[0, 1], out[0, 2])


# speedup vs baseline: 1.2937x; 1.2937x over previous
import jax
import jax.numpy as jnp
from jax.experimental import pallas as pl
from jax.experimental.pallas import tpu as pltpu

_MATCH_THRESH = 0.5
_NEG_POS = 3.0
_VAR0 = 0.1
_VAR1 = 0.2


def _per_image_kernel(targets_ref, anchors_ref, pred_loc_ref, pred_conf_ref,
                      ce_ref, stats_ref):
    tgt = targets_ref[0]
    tx0 = tgt[:, 0:1]
    ty0 = tgt[:, 1:2]
    tx1 = tgt[:, 2:3]
    ty1 = tgt[:, 3:4]
    tlab = tgt[:, 4:5]

    acx = anchors_ref[0, 0:1, :]
    acy = anchors_ref[0, 1:2, :]
    aw = anchors_ref[0, 2:3, :]
    ah = anchors_ref[0, 3:4, :]
    px0 = acx - aw * 0.5
    py0 = acy - ah * 0.5
    px1 = acx + aw * 0.5
    py1 = acy + ah * 0.5

    ix = jnp.minimum(tx1, px1) - jnp.maximum(tx0, px0)
    iy = jnp.minimum(ty1, py1) - jnp.maximum(ty0, py0)
    inter = jnp.maximum(ix, 0.0) * jnp.maximum(iy, 0.0)
    area_t = (tx1 - tx0) * (ty1 - ty0)
    area_p = (px1 - px0) * (py1 - py0)
    ovl = inter / (area_t + area_p - inter)

    n_t, A = ovl.shape
    j_iota = jax.lax.broadcasted_iota(jnp.int32, (n_t, A), 0)
    a_iota = jax.lax.broadcasted_iota(jnp.int32, (n_t, A), 1)
    lane_a = jax.lax.broadcasted_iota(jnp.int32, (1, A), 1)

    bto = jnp.max(ovl, axis=0, keepdims=True)
    bti = jnp.min(jnp.where(ovl == bto, j_iota, n_t),
                  axis=0, keepdims=True)
    rmax = jnp.max(ovl, axis=1, keepdims=True)
    bpi = jnp.min(jnp.where(ovl == rmax, a_iota, A),
                  axis=1, keepdims=True)

    m_claim = lane_a == bpi
    chosen = jnp.max(jnp.where(m_claim, j_iota, -1), axis=0,
                     keepdims=True)
    claimed = chosen >= 0
    bto = jnp.where(claimed, 2.0, bto)
    bti = jnp.where(claimed, chosen, bti)

    m_sel = (j_iota == bti).astype(jnp.float32)
    tgt_t = jnp.transpose(tgt, (1, 0))
    matched = jax.lax.dot_general(
        tgt_t, m_sel, (((1,), (0,)), ((), ())),
        preferred_element_type=jnp.float32)
    mx0 = matched[0:1, :]
    my0 = matched[1:2, :]
    mx1 = matched[2:3, :]
    my1 = matched[3:4, :]
    mlab = matched[4:5, :]
    conf = jnp.where(bto < _MATCH_THRESH, 0.0, mlab + 1.0)
    pos = conf > 0.0
    pf = pos.astype(jnp.float32)

    gcx = ((mx0 + mx1) * 0.5 - acx) / (_VAR0 * aw)
    gcy = ((my0 + my1) * 0.5 - acy) / (_VAR0 * ah)
    gw = jnp.log((mx1 - mx0) / aw) / _VAR1
    gh = jnp.log((my1 - my0) / ah) / _VAR1

    lloss = jnp.float32(0.0)
    for i, gc in enumerate((gcx, gcy, gw, gh)):
        d = pred_loc_ref[0, i:i + 1, :] - gc
        ad = jnp.abs(d)
        sl1 = jnp.where(ad < 1.0, 0.5 * d * d, ad - 0.5)
        lloss = lloss + jnp.sum(sl1 * pf)
    npos = jnp.sum(pf)

    xt = pred_conf_ref[:, pl.program_id(0) % 8, :]
    C = xt.shape[0]
    se_row = jnp.sum(jnp.exp(xt), axis=0, keepdims=True)
    cls_sub = jax.lax.broadcasted_iota(jnp.int32, xt.shape, 0)
    conf_i = conf.astype(jnp.int32)
    xg_row = jnp.sum(jnp.where(cls_sub == conf_i, xt, 0.0),
                     axis=0, keepdims=True)
    ce_row = jnp.log(se_row) - xg_row

    pos_sum = jnp.sum(jnp.where(pos, ce_row, 0.0))
    ce_ref[...] = jnp.where(pos, 0.0, ce_row).reshape(1, 1, A)

    lane128 = jax.lax.broadcasted_iota(jnp.int32, (1, 128), 1)
    stats = jnp.where(lane128 == 0, npos,
                      jnp.where(lane128 == 1, pos_sum,
                                jnp.where(lane128 == 2, lloss, 0.0)))
    stats_ref[...] = stats.reshape(1, 1, 128)


def _finalize_kernel(ce_ref, stats_ref, out_ref):
    ce = ce_ref[:, 0, :]
    stats = stats_ref[:, 0, :]
    npos = stats[:, 0:1]
    pos_sum = stats[:, 1:2]
    lloss = stats[:, 2:3]
    A = ce.shape[1]

    k = jnp.maximum(10.0, jnp.minimum(npos * _NEG_POS, A - npos))

    bits = jax.lax.bitcast_convert_type(ce, jnp.int32)
    key = jnp.where(bits < 0, jnp.int32(-2147483648) - bits, bits)

    lo = jnp.min(key, axis=1, keepdims=True) - 1
    hi = jnp.max(key, axis=1, keepdims=True)

    CH = 128
    nfull = A // CH
    ntail = A - nfull * CH

    def _tree_sum(sel):
        pieces = [sel[:, i * CH:(i + 1) * CH] for i in range(nfull)]
        while len(pieces) > 1:
            nxt = [pieces[i] + pieces[i + 1]
                   for i in range(0, len(pieces) - 1, 2)]
            if len(pieces) % 2:
                nxt.append(pieces[-1])
            pieces = nxt
        total = jnp.sum(pieces[0], axis=1, keepdims=True)
        if ntail:
            total = total + jnp.sum(sel[:, nfull * CH:], axis=1,
                                    keepdims=True)
        return total

    def count_gt(mid):
        return _tree_sum(jnp.where(key > mid, 1.0, 0.0))

    for _ in range(32):
        mid = lo + ((hi - lo) >> 1)
        c = count_gt(mid)
        ge = c >= k
        lo = jnp.where(ge, mid, lo)
        hi = jnp.where(ge, hi, mid)
    cnt = count_gt(hi)
    s = _tree_sum(jnp.where(key > hi, ce, 0.0))
    vkb = jnp.where(hi < 0, jnp.int32(-2147483648) - hi, hi)
    vk = jax.lax.bitcast_convert_type(vkb, jnp.float32)
    neg_sum = s + (k - cnt) * vk

    class_loss = jnp.sum(pos_sum + neg_sum)
    n_total = jnp.sum(npos)
    l_total = jnp.sum(lloss)

    lane128 = jax.lax.broadcasted_iota(jnp.int32, (1, 128), 1)
    out = jnp.where(lane128 == 0, class_loss / n_total,
                    jnp.where(lane128 == 1, l_total / n_total,
                              jnp.where(lane128 == 2, n_total, 0.0)))
    out_ref[...] = out


def kernel(pred_conf, pred_loc, anchors, targets):
    B, A, C = pred_conf.shape
    n_obj = targets.shape[1]
    anchors_t = jnp.transpose(anchors, (0, 2, 1))
    pred_loc_t = jnp.transpose(pred_loc, (0, 2, 1))
    pred_conf_t = jnp.transpose(pred_conf, (2, 0, 1))

    ce_neg, stats = pl.pallas_call(
        _per_image_kernel,
        grid=(B,),
        in_specs=[
            pl.BlockSpec((1, n_obj, 5), lambda b: (b, 0, 0)),
            pl.BlockSpec((1, 4, A), lambda b: (0, 0, 0)),
            pl.BlockSpec((1, 4, A), lambda b: (b, 0, 0)),
            pl.BlockSpec((C, 8, A), lambda b: (0, b // 8, 0)),
        ],
        out_specs=[
            pl.BlockSpec((1, 1, A), lambda b: (b, 0, 0)),
            pl.BlockSpec((1, 1, 128), lambda b: (b, 0, 0)),
        ],
        out_shape=[
            jax.ShapeDtypeStruct((B, 1, A), jnp.float32),
            jax.ShapeDtypeStruct((B, 1, 128), jnp.float32),
        ],
    )(targets, anchors_t, pred_loc_t, pred_conf_t)

    out = pl.pallas_call(
        _finalize_kernel,
        out_shape=jax.ShapeDtypeStruct((1, 128), jnp.float32),
    )(ce_neg, stats)

    return (out[0, 0], out[0, 1], out[0, 2])
